# baseline (speedup 1.0000x reference)
import math

import jax
import jax.numpy as jnp
from jax import lax
from jax.experimental import pallas as pl
from jax.experimental.pallas import tpu as pltpu
from jax.experimental.pallas import tpu_sc as plsc

D_MODEL = 512
LANES = 16
NUM_CORES = 2
NUM_SUBCORES = 16
NUM_WORKERS = NUM_CORES * NUM_SUBCORES
SCALE = math.sqrt(D_MODEL)

CHUNK = 32
NBUF = 4


def _scale_rows(buf_v):
    def row_body(r, carry):
        for g in range(D_MODEL // LANES):
            sl = pl.ds(g * LANES, LANES)
            buf_v[r, sl] = buf_v[r, sl] * SCALE
        return carry

    lax.fori_loop(0, buf_v.shape[0], row_body, 0)


def _emb_body(idx_hbm, table_hbm, out_hbm, idx_v, *scratch):
    bufs = scratch[:NBUF]
    gsems = scratch[NBUF : 2 * NBUF]
    osems = scratch[2 * NBUF :]

    cid = lax.axis_index("c")
    sid = lax.axis_index("s")
    wid = cid * NUM_SUBCORES + sid

    seq = idx_hbm.shape[1]
    cpw = idx_v.shape[0]
    n_chunks = cpw // CHUNK
    n_rounds = n_chunks // NBUF
    wpb = seq // cpw

    bat = wid // wpb
    soff = (wid % wpb) * cpw
    obase = wid * cpw

    pltpu.sync_copy(idx_hbm.at[bat, pl.ds(soff, cpw)], idx_v)

    def gather(ch, b):
        return pltpu.async_copy(
            table_hbm.at[idx_v.at[pl.ds(ch * CHUNK, CHUNK)]], bufs[b], gsems[b]
        )

    def out_copy(ch, b):
        return pltpu.async_copy(
            bufs[b], out_hbm.at[pl.ds(obase + ch * CHUNK, CHUNK)], osems[b]
        )

    for b in range(NBUF):
        gather(b, b)

    def round_body(r, carry):
        base = r * NBUF
        for b in range(NBUF):
            ch = base + b
            pltpu.make_async_copy(
                table_hbm.at[idx_v.at[pl.ds(0, CHUNK)]], bufs[b], gsems[b]
            ).wait()
            _scale_rows(bufs[b])
            out_copy(ch, b)

            @pl.when(r < n_rounds - 1)
            def _():
                pltpu.make_async_copy(
                    bufs[b], out_hbm.at[pl.ds(0, CHUNK)], osems[b]
                ).wait()
                gather(ch + NBUF, b)

        return carry

    lax.fori_loop(0, n_rounds, round_body, 0)

    for b in range(NBUF):
        pltpu.make_async_copy(bufs[b], out_hbm.at[pl.ds(0, CHUNK)], osems[b]).wait()


def kernel(x, table):
    b, s = x.shape
    n_total = b * s
    assert n_total % (NUM_WORKERS * CHUNK) == 0
    idx = x.astype(jnp.int32)

    mesh = plsc.VectorSubcoreMesh(
        core_axis_name="c",
        subcore_axis_name="s",
        num_cores=NUM_CORES,
        num_subcores=NUM_SUBCORES,
    )
    out = pl.kernel(
        _emb_body,
        out_type=jax.ShapeDtypeStruct((n_total, D_MODEL), jnp.float32),
        mesh=mesh,
        scratch_types=(
            [pltpu.VMEM((n_total // NUM_WORKERS,), jnp.int32)]
            + [pltpu.VMEM((CHUNK, D_MODEL), jnp.float32)] * NBUF
            + [pltpu.SemaphoreType.DMA] * (2 * NBUF)
        ),
    )(idx, table)
    return out.reshape(b, s, D_MODEL)


# speedup vs baseline: 1.0315x; 1.0315x over previous
import math

import jax
import jax.numpy as jnp
from jax import lax
from jax.experimental import pallas as pl
from jax.experimental.pallas import tpu as pltpu
from jax.experimental.pallas import tpu_sc as plsc

D_MODEL = 512
LANES = 16
NUM_CORES = 2
NUM_SUBCORES = 16
NUM_WORKERS = NUM_CORES * NUM_SUBCORES
SCALE = math.sqrt(D_MODEL)

CHUNK = 16
NBUF = 8


def _scale_rows(buf_v):
    def row_body(r, carry):
        for g in range(D_MODEL // LANES):
            sl = pl.ds(g * LANES, LANES)
            buf_v[r, sl] = buf_v[r, sl] * SCALE
        return carry

    lax.fori_loop(0, buf_v.shape[0], row_body, 0)


def _emb_body(idx_hbm, table_hbm, out_hbm, idx_v, *scratch):
    bufs = scratch[:NBUF]
    gsems = scratch[NBUF : 2 * NBUF]
    osems = scratch[2 * NBUF :]

    cid = lax.axis_index("c")
    sid = lax.axis_index("s")
    wid = cid * NUM_SUBCORES + sid

    seq = idx_hbm.shape[1]
    cpw = idx_v.shape[0]
    n_chunks = cpw // CHUNK
    n_rounds = n_chunks // NBUF
    wpb = seq // cpw

    bat = wid // wpb
    soff = (wid % wpb) * cpw
    obase = wid * cpw

    pltpu.sync_copy(idx_hbm.at[bat, pl.ds(soff, cpw)], idx_v)

    def gather(ch, b):
        return pltpu.async_copy(
            table_hbm.at[idx_v.at[pl.ds(ch * CHUNK, CHUNK)]], bufs[b], gsems[b]
        )

    def out_copy(ch, b):
        return pltpu.async_copy(
            bufs[b], out_hbm.at[pl.ds(obase + ch * CHUNK, CHUNK)], osems[b]
        )

    for b in range(NBUF):
        gather(b, b)

    def round_body(r, carry):
        base = r * NBUF
        for b in range(NBUF):
            ch = base + b
            pltpu.make_async_copy(
                table_hbm.at[idx_v.at[pl.ds(0, CHUNK)]], bufs[b], gsems[b]
            ).wait()
            _scale_rows(bufs[b])
            out_copy(ch, b)

            @pl.when(r < n_rounds - 1)
            def _():
                pltpu.make_async_copy(
                    bufs[b], out_hbm.at[pl.ds(0, CHUNK)], osems[b]
                ).wait()
                gather(ch + NBUF, b)

        return carry

    lax.fori_loop(0, n_rounds, round_body, 0)

    for b in range(NBUF):
        pltpu.make_async_copy(bufs[b], out_hbm.at[pl.ds(0, CHUNK)], osems[b]).wait()


def kernel(x, table):
    b, s = x.shape
    n_total = b * s
    assert n_total % (NUM_WORKERS * CHUNK) == 0
    idx = x.astype(jnp.int32)

    mesh = plsc.VectorSubcoreMesh(
        core_axis_name="c",
        subcore_axis_name="s",
        num_cores=NUM_CORES,
        num_subcores=NUM_SUBCORES,
    )
    out = pl.kernel(
        _emb_body,
        out_type=jax.ShapeDtypeStruct((n_total, D_MODEL), jnp.float32),
        mesh=mesh,
        scratch_types=(
            [pltpu.VMEM((n_total // NUM_WORKERS,), jnp.int32)]
            + [pltpu.VMEM((CHUNK, D_MODEL), jnp.float32)] * NBUF
            + [pltpu.SemaphoreType.DMA] * (2 * NBUF)
        ),
    )(idx, table)
    return out.reshape(b, s, D_MODEL)
